# baseline (speedup 1.0000x reference)
import functools

import jax
import jax.numpy as jnp
from jax.experimental import pallas as pl
from jax.experimental.pallas import tpu as pltpu

_UNIT = 2048
_NUNIT = 4
_TB = _UNIT * _NUNIT
_NBUF = 6


def _compute(x_blk, w1_v, b1_v, w2_v, b2_v):
    xb = x_blk.astype(jnp.bfloat16)
    h = jnp.dot(xb, w1_v[...].astype(jnp.bfloat16),
                preferred_element_type=jnp.float32)
    h = jnp.maximum(h + b1_v[...], 0.0)
    y = jnp.dot(h.astype(jnp.bfloat16), w2_v[...].astype(jnp.bfloat16),
                preferred_element_type=jnp.float32)
    return y + b2_v[...]


def _ring_kernel(x_hbm, w1_any, b1_any, w2_any, b2_any, out_hbm,
                 xbuf, ybuf, w1_v, b1_v, w2_v, b2_v,
                 rsem, wsem, wtsem, *, nsteps):
    i = pl.program_id(0)
    slot = jax.lax.rem(i, _NBUF)

    def start_read(j, s, u):
        pltpu.make_async_copy(
            x_hbm.at[pl.ds(j * _TB + u * _UNIT, _UNIT), :],
            xbuf.at[s, pl.ds(u * _UNIT, _UNIT)], rsem.at[s, u]).start()

    def wait_read(s, u):
        pltpu.make_async_copy(
            x_hbm.at[pl.ds(0, _UNIT), :],
            xbuf.at[s, pl.ds(u * _UNIT, _UNIT)], rsem.at[s, u]).wait()

    def start_write(j, s, u):
        pltpu.make_async_copy(
            ybuf.at[s, pl.ds(u * _UNIT, _UNIT)],
            out_hbm.at[pl.ds(j * _TB + u * _UNIT, _UNIT), :],
            wsem.at[s, u]).start()

    def wait_write(s, u):
        pltpu.make_async_copy(
            ybuf.at[s, pl.ds(u * _UNIT, _UNIT)],
            out_hbm.at[pl.ds(0, _UNIT), :], wsem.at[s, u]).wait()

    @pl.when(i == 0)
    def _prologue():
        pltpu.make_async_copy(w1_any, w1_v, wtsem.at[0]).start()
        pltpu.make_async_copy(b1_any, b1_v, wtsem.at[1]).start()
        pltpu.make_async_copy(w2_any, w2_v, wtsem.at[2]).start()
        pltpu.make_async_copy(b2_any, b2_v, wtsem.at[3]).start()
        for j in range(min(_NBUF - 1, nsteps)):
            for u in range(_NUNIT):
                start_read(j, j, u)
        pltpu.make_async_copy(w1_any, w1_v, wtsem.at[0]).wait()
        pltpu.make_async_copy(b1_any, b1_v, wtsem.at[1]).wait()
        pltpu.make_async_copy(w2_any, w2_v, wtsem.at[2]).wait()
        pltpu.make_async_copy(b2_any, b2_v, wtsem.at[3]).wait()

    @pl.when(i + _NBUF - 1 < nsteps)
    def _read_ahead():
        j = i + _NBUF - 1
        s = jax.lax.rem(j, _NBUF)
        for u in range(_NUNIT):
            start_read(j, s, u)

    for u in range(_NUNIT):
        wait_read(slot, u)
        @pl.when(i >= _NBUF)
        def _drain(u=u):
            wait_write(slot, u)
        ybuf[slot, pl.ds(u * _UNIT, _UNIT)] = _compute(
            xbuf[slot, pl.ds(u * _UNIT, _UNIT)], w1_v, b1_v, w2_v, b2_v)
        start_write(i, slot, u)

    @pl.when(i == nsteps - 1)
    def _epilogue():
        ntail = min(_NBUF, nsteps)
        for d in range(ntail):
            j = nsteps - ntail + d
            for u in range(_NUNIT):
                wait_write(jax.lax.rem(jnp.int32(j), _NBUF), u)


def _auto_kernel(x_ref, w1_ref, b1_ref, w2_ref, b2_ref, out_ref):
    out_ref[...] = _compute(x_ref[...], w1_ref, b1_ref, w2_ref, b2_ref)


def kernel(x, w1, b1, w2, b2):
    B, in_dim = x.shape
    hid = w1.shape[1]
    out_dim = w2.shape[1]

    flops = 2 * B * (in_dim * hid + hid * out_dim)
    bytes_accessed = 4 * (B * in_dim + B * out_dim) + 4 * (
        in_dim * hid + hid + hid * out_dim + out_dim)
    cost = pl.CostEstimate(flops=flops, transcendentals=0,
                           bytes_accessed=bytes_accessed)
    out_shape = jax.ShapeDtypeStruct((B, out_dim), jnp.float32)

    if B % _TB == 0 and B // _TB >= 2:
        nsteps = B // _TB
        return pl.pallas_call(
            functools.partial(_ring_kernel, nsteps=nsteps),
            out_shape=out_shape,
            grid=(nsteps,),
            in_specs=[pl.BlockSpec(memory_space=pl.ANY)] * 5,
            out_specs=pl.BlockSpec(memory_space=pl.ANY),
            scratch_shapes=[
                pltpu.VMEM((_NBUF, _TB, in_dim), jnp.float32),
                pltpu.VMEM((_NBUF, _TB, out_dim), jnp.float32),
                pltpu.VMEM((in_dim, hid), jnp.float32),
                pltpu.VMEM((1, hid), jnp.float32),
                pltpu.VMEM((hid, out_dim), jnp.float32),
                pltpu.VMEM((1, out_dim), jnp.float32),
                pltpu.SemaphoreType.DMA((_NBUF, _NUNIT)),
                pltpu.SemaphoreType.DMA((_NBUF, _NUNIT)),
                pltpu.SemaphoreType.DMA((4,)),
            ],
            compiler_params=pltpu.CompilerParams(
                dimension_semantics=("arbitrary",)),
            cost_estimate=cost,
        )(x, w1, b1, w2, b2)

    tb = min(16384, B)
    return pl.pallas_call(
        _auto_kernel,
        out_shape=out_shape,
        grid=(pl.cdiv(B, tb),),
        in_specs=[
            pl.BlockSpec((tb, in_dim), lambda i: (i, 0)),
            pl.BlockSpec((in_dim, hid), lambda i: (0, 0)),
            pl.BlockSpec((1, hid), lambda i: (0, 0)),
            pl.BlockSpec((hid, out_dim), lambda i: (0, 0)),
            pl.BlockSpec((1, out_dim), lambda i: (0, 0)),
        ],
        out_specs=pl.BlockSpec((tb, out_dim), lambda i: (i, 0)),
        compiler_params=pltpu.CompilerParams(
            dimension_semantics=("arbitrary",)),
        cost_estimate=cost,
    )(x, w1, b1, w2, b2)


# speedup vs baseline: 1.0315x; 1.0315x over previous
import functools

import jax
import jax.numpy as jnp
from jax.experimental import pallas as pl
from jax.experimental.pallas import tpu as pltpu

_TB = 8192
_NBUF = 6


def _compute(x_blk, w1_v, b1_v, w2_v, b2_v):
    xb = x_blk.astype(jnp.bfloat16)
    h = jnp.dot(xb, w1_v[...].astype(jnp.bfloat16),
                preferred_element_type=jnp.float32)
    h = jnp.maximum(h + b1_v[...], 0.0)
    y = jnp.dot(h.astype(jnp.bfloat16), w2_v[...].astype(jnp.bfloat16),
                preferred_element_type=jnp.float32)
    return y + b2_v[...]


def _ring_kernel(x_hbm, w1_any, b1_any, w2_any, b2_any, out_hbm,
                 xbuf, ybuf, w1_v, b1_v, w2_v, b2_v,
                 rsem, wsem, wtsem, *, tb, nsteps):
    i = pl.program_id(0)
    slot = jax.lax.rem(i, _NBUF)

    def start_read(j, s):
        pltpu.make_async_copy(
            x_hbm.at[pl.ds(j * tb, tb), :], xbuf.at[s], rsem.at[s]).start()

    def wait_read(s):
        pltpu.make_async_copy(
            x_hbm.at[pl.ds(0, tb), :], xbuf.at[s], rsem.at[s]).wait()

    def start_write(j, s):
        pltpu.make_async_copy(
            ybuf.at[s], out_hbm.at[pl.ds(j * tb, tb), :], wsem.at[s]).start()

    def wait_write(s):
        pltpu.make_async_copy(
            ybuf.at[s], out_hbm.at[pl.ds(0, tb), :], wsem.at[s]).wait()

    @pl.when(i == 0)
    def _prologue():
        pltpu.make_async_copy(w1_any, w1_v, wtsem.at[0]).start()
        pltpu.make_async_copy(b1_any, b1_v, wtsem.at[1]).start()
        pltpu.make_async_copy(w2_any, w2_v, wtsem.at[2]).start()
        pltpu.make_async_copy(b2_any, b2_v, wtsem.at[3]).start()
        for j in range(_NBUF - 1):
            if j < nsteps:
                start_read(j, j)
        pltpu.make_async_copy(w1_any, w1_v, wtsem.at[0]).wait()
        pltpu.make_async_copy(b1_any, b1_v, wtsem.at[1]).wait()
        pltpu.make_async_copy(w2_any, w2_v, wtsem.at[2]).wait()
        pltpu.make_async_copy(b2_any, b2_v, wtsem.at[3]).wait()

    @pl.when(i + _NBUF - 1 < nsteps)
    def _read_ahead():
        j = i + _NBUF - 1
        start_read(j, jax.lax.rem(j, _NBUF))

    wait_read(slot)

    @pl.when(i >= _NBUF)
    def _drain():
        wait_write(slot)

    ybuf[slot] = _compute(xbuf[slot], w1_v, b1_v, w2_v, b2_v)
    start_write(i, slot)

    @pl.when(i == nsteps - 1)
    def _epilogue():
        for d in range(min(_NBUF, nsteps)):
            j = nsteps - min(_NBUF, nsteps) + d
            wait_write(jax.lax.rem(jnp.int32(j), _NBUF))


def _auto_kernel(x_ref, w1_ref, b1_ref, w2_ref, b2_ref, out_ref):
    out_ref[...] = _compute(x_ref[...], w1_ref, b1_ref, w2_ref, b2_ref)


def kernel(x, w1, b1, w2, b2):
    B, in_dim = x.shape
    hid = w1.shape[1]
    out_dim = w2.shape[1]

    flops = 2 * B * (in_dim * hid + hid * out_dim)
    bytes_accessed = 4 * (B * in_dim + B * out_dim) + 4 * (
        in_dim * hid + hid + hid * out_dim + out_dim)
    cost = pl.CostEstimate(flops=flops, transcendentals=0,
                           bytes_accessed=bytes_accessed)
    out_shape = jax.ShapeDtypeStruct((B, out_dim), jnp.float32)

    if B % _TB == 0 and B // _TB >= 2:
        nsteps = B // _TB
        return pl.pallas_call(
            functools.partial(_ring_kernel, tb=_TB, nsteps=nsteps),
            out_shape=out_shape,
            grid=(nsteps,),
            in_specs=[pl.BlockSpec(memory_space=pl.ANY)] * 5,
            out_specs=pl.BlockSpec(memory_space=pl.ANY),
            scratch_shapes=[
                pltpu.VMEM((_NBUF, _TB, in_dim), jnp.float32),
                pltpu.VMEM((_NBUF, _TB, out_dim), jnp.float32),
                pltpu.VMEM((in_dim, hid), jnp.float32),
                pltpu.VMEM((1, hid), jnp.float32),
                pltpu.VMEM((hid, out_dim), jnp.float32),
                pltpu.VMEM((1, out_dim), jnp.float32),
                pltpu.SemaphoreType.DMA((_NBUF,)),
                pltpu.SemaphoreType.DMA((_NBUF,)),
                pltpu.SemaphoreType.DMA((4,)),
            ],
            compiler_params=pltpu.CompilerParams(
                dimension_semantics=("arbitrary",)),
            cost_estimate=cost,
        )(x, w1, b1, w2, b2)

    tb = min(16384, B)
    return pl.pallas_call(
        _auto_kernel,
        out_shape=out_shape,
        grid=(pl.cdiv(B, tb),),
        in_specs=[
            pl.BlockSpec((tb, in_dim), lambda i: (i, 0)),
            pl.BlockSpec((in_dim, hid), lambda i: (0, 0)),
            pl.BlockSpec((1, hid), lambda i: (0, 0)),
            pl.BlockSpec((hid, out_dim), lambda i: (0, 0)),
            pl.BlockSpec((1, out_dim), lambda i: (0, 0)),
        ],
        out_specs=pl.BlockSpec((tb, out_dim), lambda i: (i, 0)),
        compiler_params=pltpu.CompilerParams(
            dimension_semantics=("arbitrary",)),
        cost_estimate=cost,
    )(x, w1, b1, w2, b2)
